# baseline (speedup 1.0000x reference)
import functools

import jax
import jax.numpy as jnp
from jax import lax
from jax.experimental import pallas as pl
from jax.experimental.pallas import tpu as pltpu
from jax.experimental.pallas import tpu_sc as plsc

B, D1, D2 = 4, 4096, 2048
NC, NS, L = 2, 16, 16
NW = NC * NS
CPW = D2 // NW
NG = CPW // L
CHUNK = 512
NCHUNK = D1 // CHUNK


def _argmin_sc(x2):
    mesh = plsc.VectorSubcoreMesh(core_axis_name="c", subcore_axis_name="s")

    @functools.partial(
        pl.kernel,
        mesh=mesh,
        out_type=jax.ShapeDtypeStruct((B, D2), jnp.int32),
        compiler_params=pltpu.CompilerParams(use_tc_tiling_on_sc=False),
        scratch_types=[
            pltpu.VMEM((CHUNK, CPW), jnp.float32),
            pltpu.VMEM((CHUNK, CPW), jnp.float32),
            pltpu.VMEM((B * CPW,), jnp.int32),
            pltpu.SemaphoreType.DMA,
            pltpu.SemaphoreType.DMA,
        ],
    )
    def k(x_hbm, out_hbm, buf0, buf1, idx_v, sem0, sem1):
        wid = lax.axis_index("s") * NC + lax.axis_index("c")
        c0 = wid * CPW
        bufs = (buf0, buf1)
        sems = (sem0, sem1)
        total = B * NCHUNK

        def start(i):
            b, ch = divmod(i, NCHUNK)
            row0 = b * D1 + ch * CHUNK
            return pltpu.async_copy(
                x2_slice(x_hbm, row0, c0), bufs[i % 2], sems[i % 2])

        def x2_slice(ref, row0, c0):
            return ref.at[pl.ds(row0, CHUNK), pl.ds(c0, CPW)]

        handles = [None] * total
        handles[0] = start(0)
        for b in range(B):
            mins = tuple(jnp.full((L,), jnp.inf, jnp.float32) for _ in range(NG))
            idxs = tuple(jnp.zeros((L,), jnp.int32) for _ in range(NG))
            for ch in range(NCHUNK):
                i = b * NCHUNK + ch
                if i + 1 < total:
                    handles[i + 1] = start(i + 1)
                handles[i].wait()
                buf = bufs[i % 2]
                base = ch * CHUNK

                def body(r, carry, buf=buf, base=base):
                    mins, idxs = carry
                    rvec = jnp.full((L,), base + r, jnp.int32)
                    new_mins, new_idxs = [], []
                    for j in range(NG):
                        v = buf[r, pl.ds(j * L, L)]
                        m = v < mins[j]
                        new_mins.append(jnp.where(m, v, mins[j]))
                        new_idxs.append(jnp.where(m, rvec, idxs[j]))
                    return tuple(new_mins), tuple(new_idxs)

                mins, idxs = lax.fori_loop(0, CHUNK, body, (mins, idxs))
            for j in range(NG):
                idx_v[pl.ds(b * CPW + j * L, L)] = idxs[j]
        for b in range(B):
            pltpu.sync_copy(idx_v.at[pl.ds(b * CPW, CPW)],
                            out_hbm.at[b, pl.ds(c0, CPW)])

    return k(x2)


def kernel(x):
    x2 = x.reshape(B * D1, D2)
    return _argmin_sc(x2)


# speedup vs baseline: 2.2607x; 2.2607x over previous
import functools

import jax
import jax.numpy as jnp
from jax import lax
from jax.experimental import pallas as pl
from jax.experimental.pallas import tpu as pltpu
from jax.experimental.pallas import tpu_sc as plsc

B, D1, D2 = 4, 4096, 2048
NC, NS, L = 2, 16, 16
NW = NC * NS
CPB = 128
NBLK = D2 // CPB
NG = CPB // L
TPW = (B * NBLK) // NW
CHUNK = 256
NCHUNK = D1 // CHUNK


def _argmin_sc(x):
    mesh = plsc.VectorSubcoreMesh(core_axis_name="c", subcore_axis_name="s")

    @functools.partial(
        pl.kernel,
        mesh=mesh,
        out_type=jax.ShapeDtypeStruct((B * D2,), jnp.int32),
        scratch_types=[
            pltpu.VMEM((CHUNK, CPB), jnp.float32),
            pltpu.VMEM((CHUNK, CPB), jnp.float32),
            pltpu.VMEM((TPW * CPB,), jnp.int32),
            pltpu.SemaphoreType.DMA,
            pltpu.SemaphoreType.DMA,
        ],
    )
    def k(x_hbm, out_hbm, buf0, buf1, idx_v, sem0, sem1):
        wid = lax.axis_index("s") * NC + lax.axis_index("c")
        bufs = (buf0, buf1)
        sems = (sem0, sem1)
        total = TPW * NCHUNK

        def start(i):
            t, ch = divmod(i, NCHUNK)
            task = wid * TPW + t
            b = task // NBLK
            c0 = (task % NBLK) * CPB
            return pltpu.async_copy(
                x_hbm.at[b, pl.ds(ch * CHUNK, CHUNK), pl.ds(c0, CPB)],
                bufs[i % 2], sems[i % 2])

        handles = [None] * total
        handles[0] = start(0)
        for t in range(TPW):
            mins = tuple(jnp.full((L,), jnp.inf, jnp.float32) for _ in range(NG))
            idxs = tuple(jnp.zeros((L,), jnp.int32) for _ in range(NG))
            for ch in range(NCHUNK):
                i = t * NCHUNK + ch
                if i + 1 < total:
                    handles[i + 1] = start(i + 1)
                handles[i].wait()
                buf = bufs[i % 2]
                base = ch * CHUNK

                def body(r, carry, buf=buf, base=base):
                    mins, idxs = carry
                    rvec = jnp.full((L,), base + r, jnp.int32)
                    new_mins, new_idxs = [], []
                    for j in range(NG):
                        v = buf[r, pl.ds(j * L, L)]
                        m = v < mins[j]
                        new_mins.append(jnp.where(m, v, mins[j]))
                        new_idxs.append(jnp.where(m, rvec, idxs[j]))
                    return tuple(new_mins), tuple(new_idxs)

                mins, idxs = lax.fori_loop(0, CHUNK, body, (mins, idxs))
            for j in range(NG):
                idx_v[pl.ds(t * CPB + j * L, L)] = idxs[j]
        for t in range(TPW):
            task = wid * TPW + t
            b = task // NBLK
            c0 = (task % NBLK) * CPB
            pltpu.sync_copy(idx_v.at[pl.ds(t * CPB, CPB)],
                            out_hbm.at[pl.ds(b * D2 + c0, CPB)])

    return k(x)


def kernel(x):
    return _argmin_sc(x).reshape(B, D2)
